# baseline (speedup 1.0000x reference)
import functools

import jax
import jax.numpy as jnp
import numpy as np
from jax import lax
from jax.experimental import pallas as pl
from jax.experimental.pallas import tpu as pltpu
from jax.experimental.pallas import tpu_sc as plsc

N = 4096
NC = 2
NS = 16
L = 16
NW = NC * NS
RPW = N // NW
RCH = 4
JC = N // L
EPS = np.float32(1e-8)
INF = np.float32(np.inf)
BIGI = np.int32(2**30)

RTC = 1536
CR = 256

_mesh = plsc.VectorSubcoreMesh(
    core_axis_name="c", subcore_axis_name="s", num_cores=NC, num_subcores=NS
)


def _wid():
    return lax.axis_index("s") * NC + lax.axis_index("c")


def _tc_scan_body(dj_ref, di_ref, g_ref, val_ref, flat_ref, mv_s, mi_s):
    step = pl.program_id(0)

    @pl.when(step == 0)
    def _():
        mv_s[...] = jnp.full((1, N), INF, jnp.float32)
        mi_s[...] = jnp.zeros((1, N), jnp.int32)

    g = g_ref[...]
    b = dj_ref[...]
    a = jnp.transpose(di_ref[...])
    row = lax.broadcasted_iota(jnp.int32, (CR, N), 0) + step * CR
    col = lax.broadcasted_iota(jnp.int32, (CR, N), 1)
    t1 = b - g
    denom = a + b - 2.0 * g + EPS
    gamma = t1 / denom
    cost = b - gamma * t1
    cost = jnp.where(g < b, cost, b)
    cost = jnp.where(g < a, cost, a)
    cost = jnp.where(col > row, cost, INF)
    colmin = jnp.min(cost, axis=0, keepdims=True)
    rowmin = jnp.min(jnp.where(cost == colmin, row, BIGI), axis=0,
                     keepdims=True)
    pred = colmin < mv_s[...]
    mv_s[...] = jnp.where(pred, colmin, mv_s[...])
    mi_s[...] = jnp.where(pred, rowmin, mi_s[...])

    @pl.when(step == pl.num_programs(0) - 1)
    def _():
        bv = mv_s[...]
        flat = mi_s[...] * N + lax.broadcasted_iota(jnp.int32, (1, N), 1)
        m = jnp.min(bv)
        mf = jnp.min(jnp.where(bv == m, flat, BIGI))
        val_ref[...] = jnp.full((8, 128), m, jnp.float32)
        flat_ref[...] = jnp.full((8, 128), mf, jnp.int32)


_tc_scan = pl.pallas_call(
    _tc_scan_body,
    grid=(RTC // CR,),
    in_specs=[
        pl.BlockSpec((1, N), lambda s: (0, 0)),
        pl.BlockSpec((1, CR), lambda s: (0, s)),
        pl.BlockSpec((CR, N), lambda s: (s, 0)),
    ],
    out_specs=[
        pl.BlockSpec((8, 128), lambda s: (0, 0)),
        pl.BlockSpec((8, 128), lambda s: (0, 0)),
    ],
    out_shape=[
        jax.ShapeDtypeStruct((8, 128), jnp.float32),
        jax.ShapeDtypeStruct((8, 128), jnp.int32),
    ],
    scratch_shapes=[
        pltpu.VMEM((1, N), jnp.float32),
        pltpu.VMEM((1, N), jnp.int32),
    ],
)


@functools.partial(
    pl.kernel,
    out_type=jax.ShapeDtypeStruct((N,), jnp.float32),
    mesh=_mesh,
    scratch_types=[
        pltpu.VMEM((RPW, RPW), jnp.float32),
        pltpu.VMEM((RPW,), jnp.float32),
    ],
)
def _diag_stage(g_hbm, d_hbm, blk_v, d_v):
    w = _wid()
    r0 = w * RPW
    pltpu.sync_copy(g_hbm.at[pl.ds(r0, RPW), pl.ds(r0, RPW)], blk_v)
    iota = lax.iota(jnp.int32, L)

    def c_body(c, carry):
        def t_body(t, acc):
            v = blk_v[c * L + t, pl.ds(c * L, L)]
            return acc + jnp.where(iota == t, v, jnp.float32(0.0))

        acc = lax.fori_loop(0, L, t_body, jnp.zeros((L,), jnp.float32))
        d_v[pl.ds(c * L, L)] = acc
        return carry

    lax.fori_loop(0, RPW // L, c_body, 0)
    pltpu.sync_copy(d_v, d_hbm.at[pl.ds(r0, RPW)])


@functools.partial(
    pl.kernel,
    out_type=(
        jax.ShapeDtypeStruct((NW, L), jnp.float32),
        jax.ShapeDtypeStruct((NW, L), jnp.int32),
    ),
    mesh=_mesh,
    scratch_types=[
        pltpu.VMEM((2, RCH, N), jnp.float32),
        pltpu.VMEM((N + L,), jnp.float32),
        pltpu.VMEM((L,), jnp.float32),
        pltpu.VMEM((L,), jnp.int32),
        pltpu.SemaphoreType.DMA,
        pltpu.SemaphoreType.DMA,
    ],
)
def _scan_stage(g_hbm, d_hbm, val_hbm, flat_hbm,
                rows_v, d_v, bv_v, bf_v, sem0, sem1):
    w = _wid()
    pltpu.sync_copy(d_hbm, d_v.at[pl.ds(0, N)])
    iota = lax.iota(jnp.int32, L)
    sems = (sem0, sem1)

    HR = (N - RTC) // (2 * NW)
    bases = [RTC + w * HR + ch * RCH for ch in range(HR // RCH)]
    bases += [(N - (w + 1) * HR) + ch * RCH for ch in range(HR // RCH)]
    nchunks = len(bases)

    CB = 512

    def issue(c):
        buf = c % 2
        ri = bases[c]
        cb0 = (ri + 1) // CB

        def ibody(cb, carry):
            pltpu.async_copy(
                g_hbm.at[pl.ds(ri, RCH), pl.ds(cb * CB, CB)],
                rows_v.at[buf, :, pl.ds(cb * CB, CB)],
                sems[buf],
            )
            return carry

        lax.fori_loop(cb0, N // CB, ibody, 0)

    def drain(c):
        buf = c % 2
        ri = bases[c]
        cb0 = (ri + 1) // CB

        def dbody(cb, carry):
            pltpu.make_async_copy(
                g_hbm.at[pl.ds(ri, RCH), pl.ds(cb * CB, CB)],
                rows_v.at[buf, :, pl.ds(cb * CB, CB)],
                sems[buf],
            ).wait()
            return carry

        lax.fori_loop(cb0, N // CB, dbody, 0)

    UNR = 4

    def chunk_compute(c, carry):
        buf = c % 2
        ri = bases[c]
        q0 = (ri + 1) // (UNR * L)

        qp = jnp.minimum(q0, JC // UNR - 1)

        def row_body(rr, carry2):
            i = ri + rr
            a_vec = jnp.full((L,), d_v[pl.ds(i, L)][0], jnp.float32)
            lim = jnp.full((L,), i * (N + 1), jnp.int32)
            fl0 = i * N + qp * (UNR * L) + iota

            def one(jc, fl, sub, masked):
                bv, bf = sub
                g = rows_v[buf, rr, pl.ds(jc * L, L)]
                b = d_v[pl.ds(jc * L, L)]
                t1 = b - g
                denom = a_vec + b - 2.0 * g + EPS
                gamma = t1 / denom
                cost = b - gamma * t1
                cost = jnp.where(g < b, cost, b)
                cost = jnp.where(g < a_vec, cost, a_vec)
                pred = cost < bv
                if masked:
                    pred = pred & (fl > lim)
                bv = jnp.where(pred, cost, bv)
                bf = jnp.where(pred, fl, bf)
                return bv, bf

            subs = tuple(
                one(qp * UNR + k, fl0 + (k * L), carry2[k], True)
                for k in range(UNR)
            )
            fl1 = fl0 + (UNR * L)

            def j_body(q, carry3):
                subs, fl = carry3
                new_subs = tuple(
                    one(q * UNR + k, fl + (k * L), subs[k], False)
                    for k in range(UNR)
                )
                return new_subs, fl + (UNR * L)

            subs, _ = lax.fori_loop(qp + 1, JC // UNR, j_body, (subs, fl1))
            return subs

        return lax.fori_loop(0, RCH, row_body, carry)

    def sub_init():
        return (
            jnp.full((L,), INF, jnp.float32),
            jnp.zeros((L,), jnp.int32),
        )

    carry = tuple(sub_init() for _ in range(UNR))
    issue(0)
    for c in range(nchunks):
        if c + 1 < nchunks:
            issue(c + 1)
        drain(c)
        carry = chunk_compute(c, carry)

    bv, bf = carry[0]
    for k in range(1, UNR):
        sv, sf = carry[k]
        pred = (sv < bv) | ((sv == bv) & (sf < bf))
        bv = jnp.where(pred, sv, bv)
        bf = jnp.where(pred, sf, bf)
    bv_v[...] = bv
    bf_v[...] = bf
    pltpu.sync_copy(bv_v, val_hbm.at[w])
    pltpu.sync_copy(bf_v, flat_hbm.at[w])


@functools.partial(
    pl.kernel,
    out_type=jax.ShapeDtypeStruct((N,), jnp.float32),
    mesh=_mesh,
    scratch_types=[
        pltpu.VMEM((NW, L), jnp.float32),
        pltpu.VMEM((NW, L), jnp.int32),
        pltpu.VMEM((L,), jnp.float32),
        pltpu.VMEM((L,), jnp.int32),
        pltpu.VMEM((L,), jnp.float32),
        pltpu.VMEM((RPW,), jnp.float32),
        pltpu.VMEM((2 * L,), jnp.float32),
        pltpu.VMEM((2 * L,), jnp.int32),
        pltpu.VMEM((N + L,), jnp.float32),
        pltpu.SemaphoreType.DMA,
        pltpu.SemaphoreType.DMA,
        pltpu.SemaphoreType.DMA,
        pltpu.SemaphoreType.DMA,
    ],
)
def _merge_stage(g_hbm, d_hbm, val_hbm, flat_hbm, tcv_hbm, tcf_hbm,
                 sol_hbm,
                 cv_v, cf_v, tv_v, tf_v, dpair_v, sol_v, shv_v, shf_v,
                 grow_v, sem, semb, semc, semd):
    w = _wid()
    c1 = pltpu.async_copy(val_hbm, cv_v, semb)
    c2 = pltpu.async_copy(flat_hbm, cf_v, semc)
    c3 = pltpu.async_copy(tcv_hbm.at[0, pl.ds(0, L)], tv_v, semd)
    pltpu.sync_copy(tcf_hbm.at[0, pl.ds(0, L)], tf_v)
    c1.wait()
    c2.wait()
    c3.wait()
    iota = lax.iota(jnp.int32, L)

    def m_body(k, carry):
        bv, bf = carry
        rv = cv_v[k, pl.ds(0, L)]
        rf = cf_v[k, pl.ds(0, L)]
        pred = rv < bv
        return (
            jnp.where(pred, rv, bv),
            jnp.where(pred, rf, bf),
        )

    init = (
        jnp.full((L,), INF, jnp.float32),
        jnp.zeros((L,), jnp.int32),
    )
    bv, bf = lax.fori_loop(0, NW, m_body, init)

    tv = tv_v[...]
    tf = tf_v[...]
    predt = (tv < bv) | ((tv == bv) & (tf < bf))
    bv = jnp.where(predt, tv, bv)
    bf = jnp.where(predt, tf, bf)

    shv_v[pl.ds(L, L)] = jnp.full((L,), INF, jnp.float32)
    shf_v[pl.ds(L, L)] = jnp.full((L,), 2**30, jnp.int32)
    for k in (8, 4, 2, 1):
        shv_v[pl.ds(0, L)] = bv
        shf_v[pl.ds(0, L)] = bf
        sv = shv_v[pl.ds(k, L)]
        sf = shf_v[pl.ds(k, L)]
        pred = (sv < bv) | ((sv == bv) & (sf < bf))
        bv = jnp.where(pred, sv, bv)
        bf = jnp.where(pred, sf, bf)

    mflat = bf[0]
    i_star = lax.shift_right_logical(mflat, 12)
    j_star = lax.bitwise_and(mflat, jnp.int32(N - 1))

    pltpu.sync_copy(g_hbm.at[i_star], grow_v.at[pl.ds(0, N)])
    g_star = grow_v[pl.ds(j_star, L)][0]
    pair_idx = jnp.where(iota == 1, j_star, i_star)
    pltpu.async_copy(d_hbm.at[pair_idx], dpair_v, sem).wait()
    dp = dpair_v[...]
    a = jnp.full((L,), dp[0], jnp.float32)
    b = jnp.full((L,), dp[1], jnp.float32)
    gs = jnp.full((L,), g_star, jnp.float32)
    denom = a + b - 2.0 * gs + EPS
    gamma = (b - gs) / denom
    gamma = jnp.where(gs < b, gamma, jnp.float32(0.0))
    gamma = jnp.where(gs < a, gamma, jnp.float32(1.0))
    om_gamma = jnp.float32(1.0) - gamma

    base = w * RPW

    def s_body(c, carry):
        pos = base + c * L + iota
        v = jnp.where(pos == i_star, gamma, jnp.float32(0.0))
        v = jnp.where(pos == j_star, om_gamma, v)
        sol_v[pl.ds(c * L, L)] = v
        return carry

    lax.fori_loop(0, RPW // L, s_body, 0)
    pltpu.sync_copy(sol_v, sol_hbm.at[pl.ds(base, RPW)])


def kernel(grammian):
    d = _diag_stage(grammian)
    val, flat = _scan_stage(grammian, d)
    d_row = jnp.reshape(d, (1, N))
    tval, tflat = _tc_scan(d_row, d_row[:, :RTC], grammian)
    return _merge_stage(grammian, d, val, flat, tval, tflat)


# speedup vs baseline: 1.0713x; 1.0713x over previous
import functools

import jax
import jax.numpy as jnp
import numpy as np
from jax import lax
from jax.experimental import pallas as pl
from jax.experimental.pallas import tpu as pltpu
from jax.experimental.pallas import tpu_sc as plsc

N = 4096
NC = 2
NS = 16
L = 16
NW = NC * NS
RPW = N // NW
RCH = 4
JC = N // L
EPS = np.float32(1e-8)
INF = np.float32(np.inf)
BIGI = np.int32(2**30)

RTC = 1792
CR = 256

_mesh = plsc.VectorSubcoreMesh(
    core_axis_name="c", subcore_axis_name="s", num_cores=NC, num_subcores=NS
)


def _wid():
    return lax.axis_index("s") * NC + lax.axis_index("c")


def _tc_scan_body(dj_ref, di_ref, g_ref, val_ref, flat_ref, mv_s, mi_s):
    step = pl.program_id(0)

    @pl.when(step == 0)
    def _():
        mv_s[...] = jnp.full((1, N), INF, jnp.float32)
        mi_s[...] = jnp.zeros((1, N), jnp.int32)

    g = g_ref[...]
    b = dj_ref[...]
    a = jnp.transpose(di_ref[...])
    row = lax.broadcasted_iota(jnp.int32, (CR, N), 0) + step * CR
    col = lax.broadcasted_iota(jnp.int32, (CR, N), 1)
    t1 = b - g
    denom = a + b - 2.0 * g + EPS
    gamma = t1 / denom
    cost = b - gamma * t1
    cost = jnp.where(g < b, cost, b)
    cost = jnp.where(g < a, cost, a)
    cost = jnp.where(col > row, cost, INF)
    colmin = jnp.min(cost, axis=0, keepdims=True)
    rowmin = jnp.min(jnp.where(cost == colmin, row, BIGI), axis=0,
                     keepdims=True)
    pred = colmin < mv_s[...]
    mv_s[...] = jnp.where(pred, colmin, mv_s[...])
    mi_s[...] = jnp.where(pred, rowmin, mi_s[...])

    @pl.when(step == pl.num_programs(0) - 1)
    def _():
        bv = mv_s[...]
        flat = mi_s[...] * N + lax.broadcasted_iota(jnp.int32, (1, N), 1)
        m = jnp.min(bv)
        mf = jnp.min(jnp.where(bv == m, flat, BIGI))
        val_ref[...] = jnp.full((8, 128), m, jnp.float32)
        flat_ref[...] = jnp.full((8, 128), mf, jnp.int32)


_tc_scan = pl.pallas_call(
    _tc_scan_body,
    grid=(RTC // CR,),
    in_specs=[
        pl.BlockSpec((1, N), lambda s: (0, 0)),
        pl.BlockSpec((1, CR), lambda s: (0, s)),
        pl.BlockSpec((CR, N), lambda s: (s, 0)),
    ],
    out_specs=[
        pl.BlockSpec((8, 128), lambda s: (0, 0)),
        pl.BlockSpec((8, 128), lambda s: (0, 0)),
    ],
    out_shape=[
        jax.ShapeDtypeStruct((8, 128), jnp.float32),
        jax.ShapeDtypeStruct((8, 128), jnp.int32),
    ],
    scratch_shapes=[
        pltpu.VMEM((1, N), jnp.float32),
        pltpu.VMEM((1, N), jnp.int32),
    ],
)


@functools.partial(
    pl.kernel,
    out_type=jax.ShapeDtypeStruct((N,), jnp.float32),
    mesh=_mesh,
    scratch_types=[
        pltpu.VMEM((RPW, RPW), jnp.float32),
        pltpu.VMEM((RPW,), jnp.float32),
    ],
)
def _diag_stage(g_hbm, d_hbm, blk_v, d_v):
    w = _wid()
    r0 = w * RPW
    pltpu.sync_copy(g_hbm.at[pl.ds(r0, RPW), pl.ds(r0, RPW)], blk_v)
    iota = lax.iota(jnp.int32, L)

    def c_body(c, carry):
        def t_body(t, acc):
            v = blk_v[c * L + t, pl.ds(c * L, L)]
            return acc + jnp.where(iota == t, v, jnp.float32(0.0))

        acc = lax.fori_loop(0, L, t_body, jnp.zeros((L,), jnp.float32))
        d_v[pl.ds(c * L, L)] = acc
        return carry

    lax.fori_loop(0, RPW // L, c_body, 0)
    pltpu.sync_copy(d_v, d_hbm.at[pl.ds(r0, RPW)])


@functools.partial(
    pl.kernel,
    out_type=(
        jax.ShapeDtypeStruct((NW, L), jnp.float32),
        jax.ShapeDtypeStruct((NW, L), jnp.int32),
    ),
    mesh=_mesh,
    scratch_types=[
        pltpu.VMEM((2, RCH, N), jnp.float32),
        pltpu.VMEM((N + L,), jnp.float32),
        pltpu.VMEM((L,), jnp.float32),
        pltpu.VMEM((L,), jnp.int32),
        pltpu.SemaphoreType.DMA,
        pltpu.SemaphoreType.DMA,
    ],
)
def _scan_stage(g_hbm, d_hbm, val_hbm, flat_hbm,
                rows_v, d_v, bv_v, bf_v, sem0, sem1):
    w = _wid()
    pltpu.sync_copy(d_hbm, d_v.at[pl.ds(0, N)])
    iota = lax.iota(jnp.int32, L)
    sems = (sem0, sem1)

    HR = (N - RTC) // (2 * NW)
    bases = [RTC + w * HR + ch * RCH for ch in range(HR // RCH)]
    bases += [(N - (w + 1) * HR) + ch * RCH for ch in range(HR // RCH)]
    nchunks = len(bases)

    CB = 512

    def issue(c):
        buf = c % 2
        ri = bases[c]
        cb0 = (ri + 1) // CB

        def ibody(cb, carry):
            pltpu.async_copy(
                g_hbm.at[pl.ds(ri, RCH), pl.ds(cb * CB, CB)],
                rows_v.at[buf, :, pl.ds(cb * CB, CB)],
                sems[buf],
            )
            return carry

        lax.fori_loop(cb0, N // CB, ibody, 0)

    def drain(c):
        buf = c % 2
        ri = bases[c]
        cb0 = (ri + 1) // CB

        def dbody(cb, carry):
            pltpu.make_async_copy(
                g_hbm.at[pl.ds(ri, RCH), pl.ds(cb * CB, CB)],
                rows_v.at[buf, :, pl.ds(cb * CB, CB)],
                sems[buf],
            ).wait()
            return carry

        lax.fori_loop(cb0, N // CB, dbody, 0)

    UNR = 4

    def chunk_compute(c, carry):
        buf = c % 2
        ri = bases[c]
        q0 = (ri + 1) // (UNR * L)

        qp = jnp.minimum(q0, JC // UNR - 1)

        def row_body(rr, carry2):
            i = ri + rr
            a_vec = jnp.full((L,), d_v[pl.ds(i, L)][0], jnp.float32)
            lim = jnp.full((L,), i * (N + 1), jnp.int32)
            fl0 = i * N + qp * (UNR * L) + iota

            def one(jc, fl, sub, masked):
                bv, bf = sub
                g = rows_v[buf, rr, pl.ds(jc * L, L)]
                b = d_v[pl.ds(jc * L, L)]
                t1 = b - g
                denom = a_vec + b - 2.0 * g + EPS
                gamma = t1 / denom
                cost = b - gamma * t1
                cost = jnp.where(g < b, cost, b)
                cost = jnp.where(g < a_vec, cost, a_vec)
                pred = cost < bv
                if masked:
                    pred = pred & (fl > lim)
                bv = jnp.where(pred, cost, bv)
                bf = jnp.where(pred, fl, bf)
                return bv, bf

            subs = tuple(
                one(qp * UNR + k, fl0 + (k * L), carry2[k], True)
                for k in range(UNR)
            )
            fl1 = fl0 + (UNR * L)

            def j_body(q, carry3):
                subs, fl = carry3
                new_subs = tuple(
                    one(q * UNR + k, fl + (k * L), subs[k], False)
                    for k in range(UNR)
                )
                return new_subs, fl + (UNR * L)

            subs, _ = lax.fori_loop(qp + 1, JC // UNR, j_body, (subs, fl1))
            return subs

        return lax.fori_loop(0, RCH, row_body, carry)

    def sub_init():
        return (
            jnp.full((L,), INF, jnp.float32),
            jnp.zeros((L,), jnp.int32),
        )

    carry = tuple(sub_init() for _ in range(UNR))
    issue(0)
    for c in range(nchunks):
        if c + 1 < nchunks:
            issue(c + 1)
        drain(c)
        carry = chunk_compute(c, carry)

    bv, bf = carry[0]
    for k in range(1, UNR):
        sv, sf = carry[k]
        pred = (sv < bv) | ((sv == bv) & (sf < bf))
        bv = jnp.where(pred, sv, bv)
        bf = jnp.where(pred, sf, bf)
    bv_v[...] = bv
    bf_v[...] = bf
    pltpu.sync_copy(bv_v, val_hbm.at[w])
    pltpu.sync_copy(bf_v, flat_hbm.at[w])


@functools.partial(
    pl.kernel,
    out_type=jax.ShapeDtypeStruct((N,), jnp.float32),
    mesh=_mesh,
    scratch_types=[
        pltpu.VMEM((NW, L), jnp.float32),
        pltpu.VMEM((NW, L), jnp.int32),
        pltpu.VMEM((L,), jnp.float32),
        pltpu.VMEM((L,), jnp.int32),
        pltpu.VMEM((L,), jnp.float32),
        pltpu.VMEM((RPW,), jnp.float32),
        pltpu.VMEM((2 * L,), jnp.float32),
        pltpu.VMEM((2 * L,), jnp.int32),
        pltpu.VMEM((N + L,), jnp.float32),
        pltpu.SemaphoreType.DMA,
        pltpu.SemaphoreType.DMA,
        pltpu.SemaphoreType.DMA,
        pltpu.SemaphoreType.DMA,
    ],
)
def _merge_stage(g_hbm, d_hbm, val_hbm, flat_hbm, tcv_hbm, tcf_hbm,
                 sol_hbm,
                 cv_v, cf_v, tv_v, tf_v, dpair_v, sol_v, shv_v, shf_v,
                 grow_v, sem, semb, semc, semd):
    w = _wid()
    c1 = pltpu.async_copy(val_hbm, cv_v, semb)
    c2 = pltpu.async_copy(flat_hbm, cf_v, semc)
    c3 = pltpu.async_copy(tcv_hbm.at[0, pl.ds(0, L)], tv_v, semd)
    pltpu.sync_copy(tcf_hbm.at[0, pl.ds(0, L)], tf_v)
    c1.wait()
    c2.wait()
    c3.wait()
    iota = lax.iota(jnp.int32, L)

    def m_body(k, carry):
        bv, bf = carry
        rv = cv_v[k, pl.ds(0, L)]
        rf = cf_v[k, pl.ds(0, L)]
        pred = rv < bv
        return (
            jnp.where(pred, rv, bv),
            jnp.where(pred, rf, bf),
        )

    init = (
        jnp.full((L,), INF, jnp.float32),
        jnp.zeros((L,), jnp.int32),
    )
    bv, bf = lax.fori_loop(0, NW, m_body, init)

    tv = tv_v[...]
    tf = tf_v[...]
    predt = (tv < bv) | ((tv == bv) & (tf < bf))
    bv = jnp.where(predt, tv, bv)
    bf = jnp.where(predt, tf, bf)

    shv_v[pl.ds(L, L)] = jnp.full((L,), INF, jnp.float32)
    shf_v[pl.ds(L, L)] = jnp.full((L,), 2**30, jnp.int32)
    for k in (8, 4, 2, 1):
        shv_v[pl.ds(0, L)] = bv
        shf_v[pl.ds(0, L)] = bf
        sv = shv_v[pl.ds(k, L)]
        sf = shf_v[pl.ds(k, L)]
        pred = (sv < bv) | ((sv == bv) & (sf < bf))
        bv = jnp.where(pred, sv, bv)
        bf = jnp.where(pred, sf, bf)

    mflat = bf[0]
    i_star = lax.shift_right_logical(mflat, 12)
    j_star = lax.bitwise_and(mflat, jnp.int32(N - 1))

    pltpu.sync_copy(g_hbm.at[i_star], grow_v.at[pl.ds(0, N)])
    g_star = grow_v[pl.ds(j_star, L)][0]
    pair_idx = jnp.where(iota == 1, j_star, i_star)
    pltpu.async_copy(d_hbm.at[pair_idx], dpair_v, sem).wait()
    dp = dpair_v[...]
    a = jnp.full((L,), dp[0], jnp.float32)
    b = jnp.full((L,), dp[1], jnp.float32)
    gs = jnp.full((L,), g_star, jnp.float32)
    denom = a + b - 2.0 * gs + EPS
    gamma = (b - gs) / denom
    gamma = jnp.where(gs < b, gamma, jnp.float32(0.0))
    gamma = jnp.where(gs < a, gamma, jnp.float32(1.0))
    om_gamma = jnp.float32(1.0) - gamma

    base = w * RPW

    def s_body(c, carry):
        pos = base + c * L + iota
        v = jnp.where(pos == i_star, gamma, jnp.float32(0.0))
        v = jnp.where(pos == j_star, om_gamma, v)
        sol_v[pl.ds(c * L, L)] = v
        return carry

    lax.fori_loop(0, RPW // L, s_body, 0)
    pltpu.sync_copy(sol_v, sol_hbm.at[pl.ds(base, RPW)])


def kernel(grammian):
    d = _diag_stage(grammian)
    val, flat = _scan_stage(grammian, d)
    d_row = jnp.reshape(d, (1, N))
    tval, tflat = _tc_scan(d_row, d_row[:, :RTC], grammian)
    return _merge_stage(grammian, d, val, flat, tval, tflat)
